# baseline (device time: 32174 ns/iter reference)
import jax
import jax.numpy as jnp
from jax import lax
from jax.experimental import pallas as pl
from jax.experimental.pallas import tpu as pltpu

N_DEV = 4
B = 2
SQ = 128
D_MODEL = 512
HQ = 4
DH = 64
SKV_LOC = 128
BLK = 64
SCALE = 0.125


def kernel(x, Wq, K_ext, V_ext, Wo):
    def body(x_ref, wq_ref, k_ref, v_ref, wo_ref, out_ref,
             kv_buf, send_sems, recv_sem):
        my = lax.axis_index("i")

        def kv_copy(t):
            return pltpu.make_async_remote_copy(
                src_ref=kv_buf,
                dst_ref=kv_buf,
                send_sem=send_sems.at[max(t - 1, 0)],
                recv_sem=recv_sem,
                device_id=(t,),
                device_id_type=pl.DeviceIdType.MESH,
            )

        barrier = pltpu.get_barrier_semaphore()

        @pl.when(my == 0)
        def _():
            for t in range(1, N_DEV):
                pl.semaphore_signal(
                    barrier, inc=1, device_id=(t,),
                    device_id_type=pl.DeviceIdType.MESH,
                )
            pl.semaphore_wait(barrier, N_DEV - 1)

        @pl.when(my != 0)
        def _():
            pl.semaphore_signal(
                barrier, inc=1, device_id=(0,),
                device_id_type=pl.DeviceIdType.MESH,
            )
            pl.semaphore_wait(barrier, 1)

        @pl.when(my == 0)
        def _():
            kv_buf[0] = k_ref[...]
            kv_buf[1] = v_ref[...]
            for t in range(1, N_DEV):
                kv_copy(t).start()

        q_proj = [
            jnp.dot(x_ref[b], wq_ref[...],
                    preferred_element_type=jnp.float32)
            for b in range(B)
        ]

        @pl.when(my != 0)
        def _():
            kv_copy(0).wait_recv()

        row_blk = lax.broadcasted_iota(jnp.int32, (SQ, SKV_LOC), 0) // BLK
        col_blk = lax.broadcasted_iota(jnp.int32, (SQ, SKV_LOC), 1) // BLK
        mask = col_blk <= row_blk

        for b in range(B):
            kb_all = kv_buf[0, b]
            vb_all = kv_buf[1, b]
            ctx_heads = []
            for h in range(HQ):
                qh = q_proj[b][:, h * DH:(h + 1) * DH]
                kh = kb_all[:, h, :]
                vh = vb_all[:, h, :]
                s = lax.dot_general(
                    qh, kh, (((1,), (1,)), ((), ())),
                    preferred_element_type=jnp.float32,
                ) * SCALE
                s = jnp.where(mask, s, -1e9)
                m = jnp.max(s, axis=-1, keepdims=True)
                w = jnp.exp(s - m)
                w = w / jnp.sum(w, axis=-1, keepdims=True)
                ctx_heads.append(
                    jnp.dot(w, vh, preferred_element_type=jnp.float32)
                )
            ctx = jnp.concatenate(ctx_heads, axis=1)
            out_ref[b] = jnp.dot(ctx, wo_ref[...],
                                 preferred_element_type=jnp.float32)

        @pl.when(my == 0)
        def _():
            for t in range(1, N_DEV):
                kv_copy(t).wait_send()

    out_shape = jax.ShapeDtypeStruct((B, SQ, D_MODEL), jnp.float32)
    return pl.pallas_call(
        body,
        out_shape=out_shape,
        in_specs=[pl.BlockSpec(memory_space=pltpu.VMEM)] * 5,
        out_specs=pl.BlockSpec(memory_space=pltpu.VMEM),
        scratch_shapes=[
            pltpu.VMEM((2, B, SKV_LOC, HQ, DH), jnp.float32),
            pltpu.SemaphoreType.DMA((N_DEV - 1,)),
            pltpu.SemaphoreType.DMA,
        ],
        compiler_params=pltpu.CompilerParams(collective_id=0),
    )(x, Wq, K_ext, V_ext, Wo)


# device time: 20949 ns/iter; 1.5358x vs baseline; 1.5358x over previous
import jax
import jax.numpy as jnp
from jax import lax
from jax.experimental import pallas as pl
from jax.experimental.pallas import tpu as pltpu

N_DEV = 4
B = 2
SQ = 128
D_MODEL = 512
HQ = 4
DH = 64
SKV_LOC = 128
BLK = 64
SCALE = 0.125


def kernel(x, Wq, K_ext, V_ext, Wo):
    def body(x_ref, wq_ref, k_ref, v_ref, wo_ref, out_ref,
             kv16, send_sems, recv_sem):
        my = lax.axis_index("i")

        def kv_copy(t):
            return pltpu.make_async_remote_copy(
                src_ref=kv16,
                dst_ref=kv16,
                send_sem=send_sems.at[max(t - 1, 0)],
                recv_sem=recv_sem,
                device_id=(t,),
                device_id_type=pl.DeviceIdType.MESH,
            )

        barrier = pltpu.get_barrier_semaphore()

        @pl.when(my != 0)
        def _():
            pl.semaphore_signal(
                barrier, inc=1, device_id=(0,),
                device_id_type=pl.DeviceIdType.MESH,
            )

        @pl.when(my == 0)
        def _():
            pl.semaphore_wait(barrier, N_DEV - 1)
            kv16[0] = k_ref[...].astype(jnp.bfloat16)
            kv16[1] = v_ref[...].astype(jnp.bfloat16)
            for t in range(1, N_DEV):
                kv_copy(t).start()

        q_proj = [
            jnp.dot(x_ref[b], wq_ref[...],
                    preferred_element_type=jnp.float32)
            for b in range(B)
        ]

        @pl.when(my != 0)
        def _():
            kv_copy(0).wait_recv()

        row_blk = lax.broadcasted_iota(jnp.int32, (SQ, SKV_LOC), 0) // BLK
        col_blk = lax.broadcasted_iota(jnp.int32, (SQ, SKV_LOC), 1) // BLK
        mask = col_blk <= row_blk

        for b in range(B):
            kb_all = kv16[0, b].astype(jnp.float32)
            vb_all = kv16[1, b].astype(jnp.float32)
            ctx_heads = []
            for h in range(HQ):
                qh = q_proj[b][:, h * DH:(h + 1) * DH]
                kh = kb_all[:, h, :]
                vh = vb_all[:, h, :]
                s = lax.dot_general(
                    qh, kh, (((1,), (1,)), ((), ())),
                    preferred_element_type=jnp.float32,
                ) * SCALE
                s = jnp.where(mask, s, -1e9)
                m = jnp.max(s, axis=-1, keepdims=True)
                w = jnp.exp(s - m)
                w = w / jnp.sum(w, axis=-1, keepdims=True)
                ctx_heads.append(
                    jnp.dot(w, vh, preferred_element_type=jnp.float32)
                )
            ctx = jnp.concatenate(ctx_heads, axis=1)
            out_ref[b] = jnp.dot(ctx, wo_ref[...],
                                 preferred_element_type=jnp.float32)

        @pl.when(my == 0)
        def _():
            for t in range(1, N_DEV):
                kv_copy(t).wait_send()

    out_shape = jax.ShapeDtypeStruct((B, SQ, D_MODEL), jnp.float32)
    return pl.pallas_call(
        body,
        out_shape=out_shape,
        in_specs=[pl.BlockSpec(memory_space=pltpu.VMEM)] * 5,
        out_specs=pl.BlockSpec(memory_space=pltpu.VMEM),
        scratch_shapes=[
            pltpu.VMEM((2, B, SKV_LOC, HQ, DH), jnp.bfloat16),
            pltpu.SemaphoreType.DMA((N_DEV - 1,)),
            pltpu.SemaphoreType.DMA,
        ],
        compiler_params=pltpu.CompilerParams(collective_id=0),
    )(x, Wq, K_ext, V_ext, Wo)


# device time: 15861 ns/iter; 2.0285x vs baseline; 1.3208x over previous
import jax
import jax.numpy as jnp
from jax import lax
from jax.experimental import pallas as pl
from jax.experimental.pallas import tpu as pltpu

N_DEV = 4
B = 2
SQ = 128
D_MODEL = 512
HQ = 4
DH = 64
SKV_LOC = 128
BLK = 64
SCALE = 0.125


def kernel(x, Wq, K_ext, V_ext, Wo):
    Kt = jnp.transpose(K_ext, (0, 2, 3, 1))
    Vt = jnp.transpose(V_ext, (0, 2, 3, 1))

    def body(x_ref, wq_ref, kt_ref, vt_ref, wo_ref, out_ref,
             kv16, send_sems, recv_sems):
        my = lax.axis_index("i")

        def kv_copy(i, t):
            return pltpu.make_async_remote_copy(
                src_ref=kv16.at[i],
                dst_ref=kv16.at[i],
                send_sem=send_sems.at[i * (N_DEV - 1) + max(t - 1, 0)],
                recv_sem=recv_sems.at[i],
                device_id=(t,),
                device_id_type=pl.DeviceIdType.MESH,
            )

        barrier = pltpu.get_barrier_semaphore()

        @pl.when(my != 0)
        def _():
            pl.semaphore_signal(
                barrier, inc=1, device_id=(0,),
                device_id_type=pl.DeviceIdType.MESH,
            )

        @pl.when(my == 0)
        def _():
            pl.semaphore_wait(barrier, N_DEV - 1)
            kv16[0] = kt_ref[...].astype(jnp.bfloat16)
            for t in range(1, N_DEV):
                kv_copy(0, t).start()
            kv16[1] = vt_ref[...].astype(jnp.bfloat16)
            for t in range(1, N_DEV):
                kv_copy(1, t).start()

        x2 = x_ref[...].reshape(B * SQ, D_MODEL)
        q_proj = jnp.dot(x2, wq_ref[...],
                         preferred_element_type=jnp.float32)

        row_blk = lax.broadcasted_iota(jnp.int32, (SQ, SKV_LOC), 0) // BLK
        col_blk = lax.broadcasted_iota(jnp.int32, (SQ, SKV_LOC), 1) // BLK
        mask = col_blk <= row_blk

        @pl.when(my != 0)
        def _():
            kv_copy(0, 0).wait_recv()

        weights = []
        for b in range(B):
            for h in range(HQ):
                qh = q_proj[b * SQ:(b + 1) * SQ, h * DH:(h + 1) * DH]
                kh = kv16[0, b, h].astype(jnp.float32)
                s = lax.dot_general(
                    qh, kh, (((1,), (0,)), ((), ())),
                    preferred_element_type=jnp.float32,
                ) * SCALE
                s = jnp.where(mask, s, -1e9)
                m = jnp.max(s, axis=-1, keepdims=True)
                w = jnp.exp(s - m)
                weights.append(w / jnp.sum(w, axis=-1, keepdims=True))

        @pl.when(my != 0)
        def _():
            kv_copy(1, 0).wait_recv()

        ctx_rows = []
        for b in range(B):
            ctx_heads = []
            for h in range(HQ):
                vh = kv16[1, b, h].astype(jnp.float32)
                ctx_heads.append(lax.dot_general(
                    weights[b * HQ + h], vh, (((1,), (1,)), ((), ())),
                    preferred_element_type=jnp.float32,
                ))
            ctx_rows.append(jnp.concatenate(ctx_heads, axis=1))
        ctx = jnp.concatenate(ctx_rows, axis=0)
        out = jnp.dot(ctx, wo_ref[...],
                      preferred_element_type=jnp.float32)
        out_ref[...] = out.reshape(B, SQ, D_MODEL)

        @pl.when(my == 0)
        def _():
            for i in range(2):
                for t in range(1, N_DEV):
                    kv_copy(i, t).wait_send()

    out_shape = jax.ShapeDtypeStruct((B, SQ, D_MODEL), jnp.float32)
    return pl.pallas_call(
        body,
        out_shape=out_shape,
        in_specs=[pl.BlockSpec(memory_space=pltpu.VMEM)] * 5,
        out_specs=pl.BlockSpec(memory_space=pltpu.VMEM),
        scratch_shapes=[
            pltpu.VMEM((2, B, HQ, DH, SKV_LOC), jnp.bfloat16),
            pltpu.SemaphoreType.DMA((2 * (N_DEV - 1),)),
            pltpu.SemaphoreType.DMA((2,)),
        ],
        compiler_params=pltpu.CompilerParams(collective_id=0),
    )(x, Wq, Kt, Vt, Wo)
